# Initial kernel scaffold; baseline (speedup 1.0000x reference)
import jax
import jax.numpy as jnp
from jax.experimental import pallas as pl


def kernel(messages, rbf, connectivity, W_rbf, W_up, W1, b1, W2, b2, W_final):
    raise NotImplementedError("write your pallas kernel here")


# speedup vs baseline: 1.4825x; 1.4825x over previous
import functools

import jax
import jax.numpy as jnp
from jax import lax
from jax.experimental import pallas as pl
from jax.experimental.pallas import tpu as pltpu

NBLK = 128
CH = 512


def _body(off_ref, idx_hbm, msg_hbm, rbf_hbm,
          Wrbf_ref, Wup_ref, W1_ref, b1_ref, W2_ref, b2_ref, Wf_ref,
          out_ref,
          msg_buf, rbf_buf, idx_buf, acc_ref, sems,
          *, n_edges):
    g = pl.program_id(0)
    n0 = g * NBLK
    start = off_ref[g]
    end = off_ref[g + 1]
    s0 = (start // 8) * 8
    nchunks = jnp.maximum(lax.div(end - s0 + CH - 1, CH), 0)

    acc_ref[...] = jnp.zeros_like(acc_ref)

    def chunk_start(k):
        return jnp.minimum(s0 + k * CH, n_edges - CH)

    def issue(k, slot):
        s = chunk_start(k)
        pltpu.make_async_copy(msg_hbm.at[pl.ds(s, CH), :],
                              msg_buf.at[slot], sems.at[0, slot]).start()
        pltpu.make_async_copy(rbf_hbm.at[pl.ds(s, CH), :],
                              rbf_buf.at[slot], sems.at[1, slot]).start()
        pltpu.make_async_copy(idx_hbm.at[pl.ds(s, CH), :],
                              idx_buf.at[slot], sems.at[2, slot]).start()

    def wait(slot):
        s = 0
        pltpu.make_async_copy(msg_hbm.at[pl.ds(s, CH), :],
                              msg_buf.at[slot], sems.at[0, slot]).wait()
        pltpu.make_async_copy(rbf_hbm.at[pl.ds(s, CH), :],
                              rbf_buf.at[slot], sems.at[1, slot]).wait()
        pltpu.make_async_copy(idx_hbm.at[pl.ds(s, CH), :],
                              idx_buf.at[slot], sems.at[2, slot]).wait()

    @pl.when(nchunks > 0)
    def _():
        issue(0, 0)

    lanes = lax.broadcasted_iota(jnp.int32, (1, NBLK), 1)

    def loop_body(k, _):
        slot = lax.rem(k, 2)

        @pl.when(k + 1 < nchunks)
        def _():
            issue(k + 1, 1 - slot)

        wait(slot)
        t = jnp.dot(rbf_buf[slot], Wrbf_ref[...],
                    preferred_element_type=jnp.float32)
        m = msg_buf[slot] * t
        idx = idx_buf[slot]
        s = chunk_start(k)
        rows = s + lax.broadcasted_iota(jnp.int32, (CH, 1), 0)
        fresh = rows >= (s0 + k * CH)
        onehot = jnp.where(fresh, (idx - n0 == lanes).astype(jnp.bfloat16),
                           jnp.bfloat16(0))
        acc_ref[...] += lax.dot_general(
            onehot, m.astype(jnp.bfloat16),
            (((0,), (0,)), ((), ())),
            preferred_element_type=jnp.float32)

        return 0

    lax.fori_loop(0, nchunks, loop_body, 0)

    a = acc_ref[...]
    h = jnp.dot(a, Wup_ref[...], preferred_element_type=jnp.float32)
    z = jnp.dot(h, W1_ref[...], preferred_element_type=jnp.float32) + b1_ref[...]
    h = z * jax.nn.sigmoid(z)
    z = jnp.dot(h, W2_ref[...], preferred_element_type=jnp.float32) + b2_ref[...]
    h = z * jax.nn.sigmoid(z)
    out_ref[...] = jnp.dot(h, Wf_ref[...], preferred_element_type=jnp.float32)


def _run(messages, rbf, idx, W_rbf, W_up, W1, b1, W2, b2, W_final,
         n_particles, interpret=False):
    E, EMBED = messages.shape
    D_RBF = rbf.shape[1]
    OUT = W_up.shape[1]
    NT = W_final.shape[1]
    nblocks = (n_particles + NBLK - 1) // NBLK
    n_pad = nblocks * NBLK

    boundaries = jnp.arange(0, n_pad + 1, NBLK, dtype=jnp.int32)
    offsets = jnp.searchsorted(idx, boundaries).astype(jnp.int32)
    idx2 = idx.reshape(E, 1)

    grid_spec = pltpu.PrefetchScalarGridSpec(
        num_scalar_prefetch=1,
        grid=(nblocks,),
        in_specs=[
            pl.BlockSpec(memory_space=pl.ANY),
            pl.BlockSpec(memory_space=pl.ANY),
            pl.BlockSpec(memory_space=pl.ANY),
            pl.BlockSpec((D_RBF, EMBED), lambda g, off: (0, 0)),
            pl.BlockSpec((EMBED, OUT), lambda g, off: (0, 0)),
            pl.BlockSpec((OUT, OUT), lambda g, off: (0, 0)),
            pl.BlockSpec((1, OUT), lambda g, off: (0, 0)),
            pl.BlockSpec((OUT, OUT), lambda g, off: (0, 0)),
            pl.BlockSpec((1, OUT), lambda g, off: (0, 0)),
            pl.BlockSpec((OUT, NT), lambda g, off: (0, 0)),
        ],
        out_specs=pl.BlockSpec((NBLK, NT), lambda g, off: (g, 0)),
        scratch_shapes=[
            pltpu.VMEM((2, CH, EMBED), jnp.float32),
            pltpu.VMEM((2, CH, D_RBF), jnp.float32),
            pltpu.VMEM((2, CH, 1), jnp.int32),
            pltpu.VMEM((NBLK, EMBED), jnp.float32),
            pltpu.SemaphoreType.DMA((3, 2)),
        ],
    )
    out = pl.pallas_call(
        functools.partial(_body, n_edges=E),
        grid_spec=grid_spec,
        out_shape=jax.ShapeDtypeStruct((n_pad, NT), jnp.float32),
        interpret=interpret,
    )(offsets, idx2, messages, rbf, W_rbf, W_up, W1,
      b1.reshape(1, OUT), W2, b2.reshape(1, OUT), W_final)
    return out[:n_particles]


def kernel(messages, rbf, connectivity, W_rbf, W_up, W1, b1, W2, b2, W_final):
    idx = connectivity[0]
    return _run(messages, rbf, idx, W_rbf, W_up, W1, b1, W2, b2, W_final,
                n_particles=10000)


# speedup vs baseline: 2.1288x; 1.4359x over previous
import functools

import jax
import jax.numpy as jnp
from jax import lax
from jax.experimental import pallas as pl
from jax.experimental.pallas import tpu as pltpu

NBLK = 128
CH = 1024
NBUF = 4


def _body(off_ref, idx_hbm, msg_hbm, rbf_hbm,
          Wrbf_ref, Wup_ref, W1_ref, b1_ref, W2_ref, b2_ref, Wf_ref,
          out_ref,
          msg_buf, rbf_buf, idx_buf, acc_ref, sems,
          *, n_edges):
    g = pl.program_id(0)
    n0 = g * NBLK
    start = off_ref[g]
    end = off_ref[g + 1]
    s0 = (start // 8) * 8
    nchunks = jnp.maximum(lax.div(end - s0 + CH - 1, CH), 0)

    acc_ref[...] = jnp.zeros_like(acc_ref)

    def chunk_start(k):
        return jnp.minimum(s0 + k * CH, n_edges - CH)

    def issue(k, slot):
        s = chunk_start(k)
        pltpu.make_async_copy(msg_hbm.at[pl.ds(s, CH), :],
                              msg_buf.at[slot], sems.at[0, slot]).start()
        pltpu.make_async_copy(rbf_hbm.at[pl.ds(s, CH), :],
                              rbf_buf.at[slot], sems.at[1, slot]).start()
        pltpu.make_async_copy(idx_hbm.at[pl.ds(s, CH), :],
                              idx_buf.at[slot], sems.at[2, slot]).start()

    def wait(slot):
        s = 0
        pltpu.make_async_copy(msg_hbm.at[pl.ds(s, CH), :],
                              msg_buf.at[slot], sems.at[0, slot]).wait()
        pltpu.make_async_copy(rbf_hbm.at[pl.ds(s, CH), :],
                              rbf_buf.at[slot], sems.at[1, slot]).wait()
        pltpu.make_async_copy(idx_hbm.at[pl.ds(s, CH), :],
                              idx_buf.at[slot], sems.at[2, slot]).wait()

    for kp in range(NBUF - 1):
        @pl.when(kp < nchunks)
        def _():
            issue(kp, kp)

    lanes = lax.broadcasted_iota(jnp.int32, (1, NBLK), 1)

    def loop_body(k, _):
        slot = lax.rem(k, NBUF)

        @pl.when(k + NBUF - 1 < nchunks)
        def _():
            issue(k + NBUF - 1, lax.rem(k + NBUF - 1, NBUF))

        wait(slot)
        t = jnp.dot(rbf_buf[slot], Wrbf_ref[...],
                    preferred_element_type=jnp.float32)
        m = msg_buf[slot] * t
        idx = idx_buf[slot]
        s = chunk_start(k)
        rows = s + lax.broadcasted_iota(jnp.int32, (CH, 1), 0)
        fresh = rows >= (s0 + k * CH)
        onehot = ((idx - n0 == lanes) & fresh).astype(jnp.bfloat16)
        acc_ref[...] += lax.dot_general(
            onehot, m.astype(jnp.bfloat16),
            (((0,), (0,)), ((), ())),
            preferred_element_type=jnp.float32)

        return 0

    lax.fori_loop(0, nchunks, loop_body, 0)

    a = acc_ref[...]
    h = jnp.dot(a, Wup_ref[...], preferred_element_type=jnp.float32)
    z = jnp.dot(h, W1_ref[...], preferred_element_type=jnp.float32) + b1_ref[...]
    h = z * jax.nn.sigmoid(z)
    z = jnp.dot(h, W2_ref[...], preferred_element_type=jnp.float32) + b2_ref[...]
    h = z * jax.nn.sigmoid(z)
    out_ref[...] = jnp.dot(h, Wf_ref[...], preferred_element_type=jnp.float32)


def _run(messages, rbf, idx, W_rbf, W_up, W1, b1, W2, b2, W_final,
         n_particles, interpret=False):
    E, EMBED = messages.shape
    D_RBF = rbf.shape[1]
    OUT = W_up.shape[1]
    NT = W_final.shape[1]
    nblocks = (n_particles + NBLK - 1) // NBLK
    n_pad = nblocks * NBLK

    boundaries = jnp.arange(0, n_pad + 1, NBLK, dtype=jnp.int32)
    offsets = jnp.searchsorted(idx, boundaries).astype(jnp.int32)
    idx2 = idx.reshape(E, 1)

    grid_spec = pltpu.PrefetchScalarGridSpec(
        num_scalar_prefetch=1,
        grid=(nblocks,),
        in_specs=[
            pl.BlockSpec(memory_space=pl.ANY),
            pl.BlockSpec(memory_space=pl.ANY),
            pl.BlockSpec(memory_space=pl.ANY),
            pl.BlockSpec((D_RBF, EMBED), lambda g, off: (0, 0)),
            pl.BlockSpec((EMBED, OUT), lambda g, off: (0, 0)),
            pl.BlockSpec((OUT, OUT), lambda g, off: (0, 0)),
            pl.BlockSpec((1, OUT), lambda g, off: (0, 0)),
            pl.BlockSpec((OUT, OUT), lambda g, off: (0, 0)),
            pl.BlockSpec((1, OUT), lambda g, off: (0, 0)),
            pl.BlockSpec((OUT, NT), lambda g, off: (0, 0)),
        ],
        out_specs=pl.BlockSpec((NBLK, NT), lambda g, off: (g, 0)),
        scratch_shapes=[
            pltpu.VMEM((NBUF, CH, EMBED), jnp.float32),
            pltpu.VMEM((NBUF, CH, D_RBF), jnp.float32),
            pltpu.VMEM((NBUF, CH, 1), jnp.int32),
            pltpu.VMEM((NBLK, EMBED), jnp.float32),
            pltpu.SemaphoreType.DMA((3, NBUF)),
        ],
    )
    out = pl.pallas_call(
        functools.partial(_body, n_edges=E),
        grid_spec=grid_spec,
        out_shape=jax.ShapeDtypeStruct((n_pad, NT), jnp.float32),
        interpret=interpret,
    )(offsets, idx2, messages, rbf, W_rbf, W_up, W1,
      b1.reshape(1, OUT), W2, b2.reshape(1, OUT), W_final)
    return out[:n_particles]


def kernel(messages, rbf, connectivity, W_rbf, W_up, W1, b1, W2, b2, W_final):
    idx = connectivity[0]
    return _run(messages, rbf, idx, W_rbf, W_up, W1, b1, W2, b2, W_final,
                n_particles=10000)


# speedup vs baseline: 3.0367x; 1.4265x over previous
import functools

import jax
import jax.numpy as jnp
from jax import lax
from jax.experimental import pallas as pl
from jax.experimental.pallas import tpu as pltpu

NBLK = 128
CH = 1280


def _body(wb_ref, nw_ref, idx_ref, msg_ref, rbf_ref,
          Wrbf_ref, Wup_ref, W1_ref, b1_ref, W2_ref, b2_ref, Wf_ref,
          out_ref, acc_ref, *, nch, nblocks):
    j = pl.program_id(0)

    @pl.when(j == 0)
    def _():
        acc_ref[...] = jnp.zeros_like(acc_ref)

    @pl.when(j < nch)
    def _chunk():
        t = jnp.dot(rbf_ref[...], Wrbf_ref[...],
                    preferred_element_type=jnp.float32)
        m = (msg_ref[...] * t).astype(jnp.bfloat16)
        idx = idx_ref[0]
        niota = lax.broadcasted_iota(jnp.int32, (NBLK, 1), 0)
        base0 = wb_ref[j]

        def window(i, _):
            base = base0 + i * NBLK
            onehot = (idx - base == niota).astype(jnp.bfloat16)
            acc_ref[pl.ds(base, NBLK), :] += jnp.dot(
                onehot, m, preferred_element_type=jnp.float32)
            return 0

        lax.fori_loop(0, nw_ref[j], window, 0)

    @pl.when(j >= nch)
    def _mlp():
        g = j - nch
        a = acc_ref[pl.ds(g * NBLK, NBLK), :]
        h = jnp.dot(a, Wup_ref[...], preferred_element_type=jnp.float32)
        z = jnp.dot(h, W1_ref[...],
                    preferred_element_type=jnp.float32) + b1_ref[...]
        h = z * jax.nn.sigmoid(z)
        z = jnp.dot(h, W2_ref[...],
                    preferred_element_type=jnp.float32) + b2_ref[...]
        h = z * jax.nn.sigmoid(z)
        out_ref[...] = jnp.dot(h, Wf_ref[...],
                               preferred_element_type=jnp.float32)


def _run(messages, rbf, idx, W_rbf, W_up, W1, b1, W2, b2, W_final,
         n_particles, ch=CH, interpret=False):
    E, EMBED = messages.shape
    D_RBF = rbf.shape[1]
    OUT = W_up.shape[1]
    NT = W_final.shape[1]
    nblocks = (n_particles + NBLK - 1) // NBLK
    n_pad = nblocks * NBLK
    assert E % ch == 0
    nch = E // ch
    grid = nch + nblocks

    wbase = (idx[::ch] // 8) * 8
    last = idx[ch - 1::ch]
    nwin = (last - wbase) // NBLK + 1
    idx3 = idx.reshape(nch, 1, ch)

    grid_spec = pltpu.PrefetchScalarGridSpec(
        num_scalar_prefetch=2,
        grid=(grid,),
        in_specs=[
            pl.BlockSpec((1, 1, ch),
                         lambda j, wb, nw: (jnp.minimum(j, nch - 1), 0, 0)),
            pl.BlockSpec((ch, EMBED),
                         lambda j, wb, nw: (jnp.minimum(j, nch - 1), 0)),
            pl.BlockSpec((ch, D_RBF),
                         lambda j, wb, nw: (jnp.minimum(j, nch - 1), 0)),
            pl.BlockSpec((D_RBF, EMBED), lambda j, wb, nw: (0, 0)),
            pl.BlockSpec((EMBED, OUT), lambda j, wb, nw: (0, 0)),
            pl.BlockSpec((OUT, OUT), lambda j, wb, nw: (0, 0)),
            pl.BlockSpec((1, OUT), lambda j, wb, nw: (0, 0)),
            pl.BlockSpec((OUT, OUT), lambda j, wb, nw: (0, 0)),
            pl.BlockSpec((1, OUT), lambda j, wb, nw: (0, 0)),
            pl.BlockSpec((OUT, NT), lambda j, wb, nw: (0, 0)),
        ],
        out_specs=pl.BlockSpec(
            (NBLK, NT), lambda j, wb, nw: (jnp.maximum(j - nch, 0), 0)),
        scratch_shapes=[
            pltpu.VMEM((n_pad + NBLK, EMBED), jnp.float32),
        ],
    )
    out = pl.pallas_call(
        functools.partial(_body, nch=nch, nblocks=nblocks),
        grid_spec=grid_spec,
        out_shape=jax.ShapeDtypeStruct((nblocks * NBLK, NT), jnp.float32),
        interpret=interpret,
    )(wbase.astype(jnp.int32), nwin.astype(jnp.int32),
      idx3, messages, rbf, W_rbf, W_up, W1,
      b1.reshape(1, OUT), W2, b2.reshape(1, OUT), W_final)
    return out[:n_particles]


def kernel(messages, rbf, connectivity, W_rbf, W_up, W1, b1, W2, b2, W_final):
    idx = connectivity[0]
    return _run(messages, rbf, idx, W_rbf, W_up, W1, b1, W2, b2, W_final,
                n_particles=10000)


# speedup vs baseline: 3.7686x; 1.2410x over previous
import functools

import jax
import jax.numpy as jnp
from jax import lax
from jax.experimental import pallas as pl
from jax.experimental.pallas import tpu as pltpu

NBLK = 128
CH = 2560


def _body(wb_ref, nw_ref, idx_ref, msg_ref, rbf_ref,
          Wrbf_ref, Wup_ref, W1_ref, b1_ref, W2_ref, b2_ref, Wf_ref,
          out_ref, acc_ref, *, nch, nblocks):
    j = pl.program_id(0)

    @pl.when(j == 0)
    def _():
        acc_ref[...] = jnp.zeros_like(acc_ref)

    @pl.when(j < nch)
    def _chunk():
        t = jnp.dot(rbf_ref[...], Wrbf_ref[...],
                    preferred_element_type=jnp.float32)
        m = (msg_ref[...] * t).astype(jnp.bfloat16)
        idx = idx_ref[0]
        niota = lax.broadcasted_iota(jnp.int32, (NBLK, 1), 0)
        base0 = wb_ref[j]

        def window(i, _):
            base = base0 + i * NBLK
            onehot = (idx - base == niota).astype(jnp.bfloat16)
            acc_ref[pl.ds(base, NBLK), :] += jnp.dot(
                onehot, m, preferred_element_type=jnp.float32)
            return 0

        lax.fori_loop(0, nw_ref[j], window, 0)

    @pl.when(j >= nch)
    def _mlp():
        g = j - nch
        a = acc_ref[pl.ds(g * NBLK, NBLK), :]
        h = jnp.dot(a, Wup_ref[...], preferred_element_type=jnp.float32)
        z = jnp.dot(h, W1_ref[...],
                    preferred_element_type=jnp.float32) + b1_ref[...]
        h = z * jax.nn.sigmoid(z)
        z = jnp.dot(h, W2_ref[...],
                    preferred_element_type=jnp.float32) + b2_ref[...]
        h = z * jax.nn.sigmoid(z)
        out_ref[...] = jnp.dot(h, Wf_ref[...],
                               preferred_element_type=jnp.float32)


def _run(messages, rbf, idx, W_rbf, W_up, W1, b1, W2, b2, W_final,
         n_particles, ch=CH, interpret=False):
    E, EMBED = messages.shape
    D_RBF = rbf.shape[1]
    OUT = W_up.shape[1]
    NT = W_final.shape[1]
    nblocks = (n_particles + NBLK - 1) // NBLK
    n_pad = nblocks * NBLK
    assert E % ch == 0
    nch = E // ch
    grid = nch + nblocks

    wbase = (idx[::ch] // 8) * 8
    last = idx[ch - 1::ch]
    nwin = (last - wbase) // NBLK + 1
    idx3 = idx.reshape(nch, 1, ch)

    grid_spec = pltpu.PrefetchScalarGridSpec(
        num_scalar_prefetch=2,
        grid=(grid,),
        in_specs=[
            pl.BlockSpec((1, 1, ch),
                         lambda j, wb, nw: (jnp.minimum(j, nch - 1), 0, 0)),
            pl.BlockSpec((ch, EMBED),
                         lambda j, wb, nw: (jnp.minimum(j, nch - 1), 0)),
            pl.BlockSpec((ch, D_RBF),
                         lambda j, wb, nw: (jnp.minimum(j, nch - 1), 0)),
            pl.BlockSpec((D_RBF, EMBED), lambda j, wb, nw: (0, 0)),
            pl.BlockSpec((EMBED, OUT), lambda j, wb, nw: (0, 0)),
            pl.BlockSpec((OUT, OUT), lambda j, wb, nw: (0, 0)),
            pl.BlockSpec((1, OUT), lambda j, wb, nw: (0, 0)),
            pl.BlockSpec((OUT, OUT), lambda j, wb, nw: (0, 0)),
            pl.BlockSpec((1, OUT), lambda j, wb, nw: (0, 0)),
            pl.BlockSpec((OUT, NT), lambda j, wb, nw: (0, 0)),
        ],
        out_specs=pl.BlockSpec(
            (NBLK, NT), lambda j, wb, nw: (jnp.maximum(j - nch, 0), 0)),
        scratch_shapes=[
            pltpu.VMEM((n_pad + NBLK, EMBED), jnp.float32),
        ],
    )
    out = pl.pallas_call(
        functools.partial(_body, nch=nch, nblocks=nblocks),
        grid_spec=grid_spec,
        out_shape=jax.ShapeDtypeStruct((nblocks * NBLK, NT), jnp.float32),
        interpret=interpret,
    )(wbase.astype(jnp.int32), nwin.astype(jnp.int32),
      idx3, messages, rbf, W_rbf, W_up, W1,
      b1.reshape(1, OUT), W2, b2.reshape(1, OUT), W_final)
    return out[:n_particles]


def kernel(messages, rbf, connectivity, W_rbf, W_up, W1, b1, W2, b2, W_final):
    idx = connectivity[0]
    return _run(messages, rbf, idx, W_rbf, W_up, W1, b1, W2, b2, W_final,
                n_particles=10000)
